# baseline (speedup 1.0000x reference)
import functools

import jax
import jax.numpy as jnp
import numpy as np
from jax import lax
from jax.experimental import pallas as pl
from jax.experimental.pallas import tpu as pltpu
from jax.experimental.pallas import tpu_sc as plsc

_C = 256
_NZ, _NY, _NX = 10, 100, 100
_NV = _NZ * _NY * _NX
_NVP = 101376
_X0, _X1 = -40.0, 40.0
_Y0, _Y1 = -40.0, 40.0
_Z0, _Z1 = -2.0, 6.0
_VOX = 0.8
_CW = 128


def _idx_weight_kernel(xyz_ref, conf_ref, lin_ref, wt_ref, pt_ref, *, bhw, hw):
    vv = pl.program_id(0)
    bb = pl.program_id(1)
    x = xyz_ref[0, 0:1, :]
    y = xyz_ref[0, 1:2, :]
    z = xyz_ref[0, 2:3, :]
    conf = conf_ref[0, :, :]
    valid = jnp.isfinite(x) & jnp.isfinite(y) & jnp.isfinite(z)
    valid &= conf > 0.0001
    valid &= (x >= _X0) & (x < _X1)
    valid &= (y >= _Y0) & (y < _Y1)
    valid &= (z >= _Z0) & (z < _Z1)
    x_idx = (x - _X0) / _VOX
    y_idx = (y - _Y0) / _VOX
    z_idx = (z - _Z0) / _VOX
    x0 = jnp.floor(x_idx)
    y0 = jnp.floor(y_idx)
    z0 = jnp.floor(z_idx)
    fx = jnp.clip(x_idx - x0, 0.0, 1.0)
    fy = jnp.clip(y_idx - y0, 0.0, 1.0)
    fz = jnp.clip(z_idx - z0, 0.0, 1.0)
    x0i = x0.astype(jnp.int32)
    y0i = y0.astype(jnp.int32)
    z0i = z0.astype(jnp.int32)
    lins, wts = [], []
    for dx in (0, 1):
        for dy in (0, 1):
            for dz in (0, 1):
                ix = jnp.clip(x0i + dx, 0, _NX - 1)
                iy = jnp.clip(y0i + dy, 0, _NY - 1)
                iz = jnp.clip(z0i + dz, 0, _NZ - 1)
                valid = valid & (x0i + dx >= 0) & (x0i + dx < _NX)
                valid = valid & (y0i + dy >= 0) & (y0i + dy < _NY)
                valid = valid & (z0i + dz >= 0) & (z0i + dz < _NZ)
                wx = fx if dx == 1 else 1.0 - fx
                wy = fy if dy == 1 else 1.0 - fy
                wz = fz if dz == 1 else 1.0 - fz
                w = wx * wy * wz * conf * valid.astype(jnp.float32)
                lins.append(iz * (_NY * _NX) + iy * _NX + ix)
                wts.append(w)
    lin_ref[0] = jnp.concatenate(lins, axis=0)
    wt_ref[0] = jnp.concatenate(wts, axis=0)
    base = vv * hw + bb * bhw
    pt = base + jax.lax.broadcasted_iota(jnp.int32, (8, bhw), 1)
    pt_ref[0] = pt


def _feat_t_kernel(feat_ref, out_ref):
    out_ref[0] = feat_ref[0].T


def _norm_mm_kernel(vol_ref, volw_ref, w_ref, b_ref, out_ref):
    f = vol_ref[...].astype(jnp.float32)
    wsum = volw_ref[...]
    nf = f / jnp.maximum(wsum, 1e-6)
    out = jax.lax.dot_general(
        w_ref[...], nf, (((1,), (1,)), ((), ())),
        preferred_element_type=jnp.float32)
    out_ref[...] = out + b_ref[...]


def _stage1(xyz, conf, feat):
    v, _, h4, w4 = xyz.shape[0], xyz.shape[1], xyz.shape[2], xyz.shape[3]
    hw = h4 * w4
    bhw = 1408
    nb = hw // bhw
    xyz3 = xyz.reshape(v, 3, hw)
    conf3 = conf.reshape(v, 1, hw)
    lin, wt, pt = pl.pallas_call(
        functools.partial(_idx_weight_kernel, bhw=bhw, hw=hw),
        grid=(v, nb),
        in_specs=[
            pl.BlockSpec((1, 3, bhw), lambda i, j: (i, 0, j)),
            pl.BlockSpec((1, 1, bhw), lambda i, j: (i, 0, j)),
        ],
        out_specs=[
            pl.BlockSpec((1, 8, bhw), lambda i, j: (i * nb + j, 0, 0)),
            pl.BlockSpec((1, 8, bhw), lambda i, j: (i * nb + j, 0, 0)),
            pl.BlockSpec((1, 8, bhw), lambda i, j: (i * nb + j, 0, 0)),
        ],
        out_shape=[
            jax.ShapeDtypeStruct((v * nb, 8, bhw), jnp.int32),
            jax.ShapeDtypeStruct((v * nb, 8, bhw), jnp.float32),
            jax.ShapeDtypeStruct((v * nb, 8, bhw), jnp.int32),
        ],
    )(xyz3, conf3)
    feat3 = feat.reshape(v, _C, hw)
    feat_t = pl.pallas_call(
        _feat_t_kernel,
        grid=(v, nb),
        in_specs=[pl.BlockSpec((1, _C, bhw), lambda i, j: (i, 0, j))],
        out_specs=pl.BlockSpec((1, bhw, _C), lambda i, j: (i * nb + j, 0, 0)),
        out_shape=jax.ShapeDtypeStruct((v * nb, bhw, _C), jnp.float32),
    )(feat3)
    n = v * hw
    lin8 = jnp.transpose(lin, (1, 0, 2)).reshape(8, n)
    wt8 = jnp.transpose(wt, (1, 0, 2)).reshape(8, n)
    return lin8, wt8, pt.reshape(-1), feat_t.reshape(n, _C)


def _stage3(vol, vol_w, W, bias):
    bv = 1536
    out = pl.pallas_call(
        _norm_mm_kernel,
        grid=(_NVP // bv,),
        in_specs=[
            pl.BlockSpec((bv, _C), lambda i: (i, 0)),
            pl.BlockSpec((bv, 1), lambda i: (i, 0)),
            pl.BlockSpec((_C, _C), lambda i: (0, 0)),
            pl.BlockSpec((_C, 1), lambda i: (0, 0)),
        ],
        out_specs=pl.BlockSpec((_C, bv), lambda i: (0, i)),
        out_shape=jax.ShapeDtypeStruct((_C, _NVP), jnp.float32),
    )(vol, vol_w, W, bias.reshape(_C, 1))
    return out[:, :_NV]


def _scatter_stage(lin8, wt8, feat_t):
    vol = jnp.zeros((_NVP, _C), jnp.bfloat16)
    volw = jnp.zeros((_NVP, 1), jnp.float32)
    for k in range(8):
        w = wt8[k][:, None]
        vol = vol.at[lin8[k]].add((feat_t * w).astype(jnp.bfloat16))
        volw = volw.at[lin8[k]].add(w)
    return vol, volw


def kernel(feat_1_4, xyz_1_4, conf_1_4, W, bias):
    b, t, v, c, h4, w4 = feat_1_4.shape
    feat = feat_1_4.reshape(v, c, h4, w4)
    xyz = xyz_1_4.reshape(v, 3, h4, w4)
    conf = conf_1_4.reshape(v, h4, w4)
    lin, wt, pt, feat_t = _stage1(xyz, conf, feat)
    del pt
    vol, vol_w = _scatter_stage(lin, wt, feat_t)
    out = _stage3(vol, vol_w, W, bias)
    return out.reshape(b, t, c, _NZ, _NY, _NX)


# speedup vs baseline: 1.1946x; 1.1946x over previous
import functools

import jax
import jax.numpy as jnp
from jax.experimental import pallas as pl

_C = 256
_NZ, _NY, _NX = 10, 100, 100
_NV = _NZ * _NY * _NX
_NVP = 101376
_X0, _X1 = -40.0, 40.0
_Y0, _Y1 = -40.0, 40.0
_Z0, _Z1 = -2.0, 6.0
_VOX = 0.8


def _idx_weight_kernel(xyz_ref, conf_ref, lin_ref, wt_ref, pt_ref, *, bhw, hw):
    vv = pl.program_id(0)
    bb = pl.program_id(1)
    x = xyz_ref[0, 0:1, :]
    y = xyz_ref[0, 1:2, :]
    z = xyz_ref[0, 2:3, :]
    conf = conf_ref[0, :, :]
    valid = jnp.isfinite(x) & jnp.isfinite(y) & jnp.isfinite(z)
    valid &= conf > 0.0001
    valid &= (x >= _X0) & (x < _X1)
    valid &= (y >= _Y0) & (y < _Y1)
    valid &= (z >= _Z0) & (z < _Z1)
    x_idx = (x - _X0) / _VOX
    y_idx = (y - _Y0) / _VOX
    z_idx = (z - _Z0) / _VOX
    x0 = jnp.floor(x_idx)
    y0 = jnp.floor(y_idx)
    z0 = jnp.floor(z_idx)
    fx = jnp.clip(x_idx - x0, 0.0, 1.0)
    fy = jnp.clip(y_idx - y0, 0.0, 1.0)
    fz = jnp.clip(z_idx - z0, 0.0, 1.0)
    x0i = x0.astype(jnp.int32)
    y0i = y0.astype(jnp.int32)
    z0i = z0.astype(jnp.int32)
    lins, wts = [], []
    for dx in (0, 1):
        for dy in (0, 1):
            for dz in (0, 1):
                ix = jnp.clip(x0i + dx, 0, _NX - 1)
                iy = jnp.clip(y0i + dy, 0, _NY - 1)
                iz = jnp.clip(z0i + dz, 0, _NZ - 1)
                valid = valid & (x0i + dx >= 0) & (x0i + dx < _NX)
                valid = valid & (y0i + dy >= 0) & (y0i + dy < _NY)
                valid = valid & (z0i + dz >= 0) & (z0i + dz < _NZ)
                wx = fx if dx == 1 else 1.0 - fx
                wy = fy if dy == 1 else 1.0 - fy
                wz = fz if dz == 1 else 1.0 - fz
                w = wx * wy * wz * conf * valid.astype(jnp.float32)
                lins.append(iz * (_NY * _NX) + iy * _NX + ix)
                wts.append(w)
    lin_ref[0] = jnp.concatenate(lins, axis=0)
    wt_ref[0] = jnp.concatenate(wts, axis=0)
    base = vv * hw + bb * bhw
    pt = base + jax.lax.broadcasted_iota(jnp.int32, (8, bhw), 1)
    pt_ref[0] = pt


def _feat_t_kernel(feat_ref, out_ref):
    out_ref[0] = feat_ref[0].T


def _norm_mm_kernel(vol_ref, volw_ref, w_ref, b_ref, out_ref):
    f = vol_ref[...]
    wsum = volw_ref[...]
    nf = f / jnp.maximum(wsum, 1e-6)
    out = jax.lax.dot_general(
        w_ref[...], nf, (((1,), (1,)), ((), ())),
        preferred_element_type=jnp.float32)
    out_ref[...] = out + b_ref[...]


def _stage1(xyz, conf, feat):
    v, _, h4, w4 = xyz.shape[0], xyz.shape[1], xyz.shape[2], xyz.shape[3]
    hw = h4 * w4
    bhw = 1408
    nb = hw // bhw
    xyz3 = xyz.reshape(v, 3, hw)
    conf3 = conf.reshape(v, 1, hw)
    lin, wt, pt = pl.pallas_call(
        functools.partial(_idx_weight_kernel, bhw=bhw, hw=hw),
        grid=(v, nb),
        in_specs=[
            pl.BlockSpec((1, 3, bhw), lambda i, j: (i, 0, j)),
            pl.BlockSpec((1, 1, bhw), lambda i, j: (i, 0, j)),
        ],
        out_specs=[
            pl.BlockSpec((1, 8, bhw), lambda i, j: (i * nb + j, 0, 0)),
            pl.BlockSpec((1, 8, bhw), lambda i, j: (i * nb + j, 0, 0)),
            pl.BlockSpec((1, 8, bhw), lambda i, j: (i * nb + j, 0, 0)),
        ],
        out_shape=[
            jax.ShapeDtypeStruct((v * nb, 8, bhw), jnp.int32),
            jax.ShapeDtypeStruct((v * nb, 8, bhw), jnp.float32),
            jax.ShapeDtypeStruct((v * nb, 8, bhw), jnp.int32),
        ],
    )(xyz3, conf3)
    feat3 = feat.reshape(v, _C, hw)
    feat_t = pl.pallas_call(
        _feat_t_kernel,
        grid=(v, nb),
        in_specs=[pl.BlockSpec((1, _C, bhw), lambda i, j: (i, 0, j))],
        out_specs=pl.BlockSpec((1, bhw, _C), lambda i, j: (i * nb + j, 0, 0)),
        out_shape=jax.ShapeDtypeStruct((v * nb, bhw, _C), jnp.float32),
    )(feat3)
    n = v * hw
    lin8 = jnp.transpose(lin, (1, 0, 2)).reshape(8, n)
    wt8 = jnp.transpose(wt, (1, 0, 2)).reshape(8, n)
    return lin8, wt8, pt.reshape(-1), feat_t.reshape(n, _C)


def _stage3(vol, vol_w, W, bias):
    bv = 1536
    out = pl.pallas_call(
        _norm_mm_kernel,
        grid=(_NVP // bv,),
        in_specs=[
            pl.BlockSpec((bv, _C), lambda i: (i, 0)),
            pl.BlockSpec((bv, 1), lambda i: (i, 0)),
            pl.BlockSpec((_C, _C), lambda i: (0, 0)),
            pl.BlockSpec((_C, 1), lambda i: (0, 0)),
        ],
        out_specs=pl.BlockSpec((_C, bv), lambda i: (0, i)),
        out_shape=jax.ShapeDtypeStruct((_C, _NVP), jnp.float32),
    )(vol, vol_w, W, bias.reshape(_C, 1))
    return out[:, :_NV]


def _scatter_stage(lin8, wt8, feat_t):
    cs = 32
    vols = []
    for sidx in range(_C // cs):
        vs = jnp.zeros((_NVP, cs), jnp.float32)
        fslab = feat_t[:, sidx * cs:(sidx + 1) * cs]
        for k in range(8):
            vs = vs.at[lin8[k]].add(fslab * wt8[k][:, None])
        vols.append(vs)
    vol = jnp.concatenate(vols, axis=1)
    volw = jnp.zeros((_NVP, 1), jnp.float32)
    for k in range(8):
        volw = volw.at[lin8[k]].add(wt8[k][:, None])
    return vol, volw


def kernel(feat_1_4, xyz_1_4, conf_1_4, W, bias):
    b, t, v, c, h4, w4 = feat_1_4.shape
    feat = feat_1_4.reshape(v, c, h4, w4)
    xyz = xyz_1_4.reshape(v, 3, h4, w4)
    conf = conf_1_4.reshape(v, h4, w4)
    lin, wt, pt, feat_t = _stage1(xyz, conf, feat)
    del pt
    vol, vol_w = _scatter_stage(lin, wt, feat_t)
    out = _stage3(vol, vol_w, W, bias)
    return out.reshape(b, t, c, _NZ, _NY, _NX)


# speedup vs baseline: 3.6199x; 3.0303x over previous
import functools

import jax
import jax.numpy as jnp
from jax.experimental import pallas as pl

_C = 256
_NZ, _NY, _NX = 10, 100, 100
_NV = _NZ * _NY * _NX
_NVP = 101376
_X0, _X1 = -40.0, 40.0
_Y0, _Y1 = -40.0, 40.0
_Z0, _Z1 = -2.0, 6.0
_VOX = 0.8


def _idx_weight_kernel(xyz_ref, conf_ref, lin_ref, wt_ref, pt_ref, *, bhw, hw):
    vv = pl.program_id(0)
    bb = pl.program_id(1)
    x = xyz_ref[0, 0:1, :]
    y = xyz_ref[0, 1:2, :]
    z = xyz_ref[0, 2:3, :]
    conf = conf_ref[0, :, :]
    valid = jnp.isfinite(x) & jnp.isfinite(y) & jnp.isfinite(z)
    valid &= conf > 0.0001
    valid &= (x >= _X0) & (x < _X1)
    valid &= (y >= _Y0) & (y < _Y1)
    valid &= (z >= _Z0) & (z < _Z1)
    x_idx = (x - _X0) / _VOX
    y_idx = (y - _Y0) / _VOX
    z_idx = (z - _Z0) / _VOX
    x0 = jnp.floor(x_idx)
    y0 = jnp.floor(y_idx)
    z0 = jnp.floor(z_idx)
    fx = jnp.clip(x_idx - x0, 0.0, 1.0)
    fy = jnp.clip(y_idx - y0, 0.0, 1.0)
    fz = jnp.clip(z_idx - z0, 0.0, 1.0)
    x0i = x0.astype(jnp.int32)
    y0i = y0.astype(jnp.int32)
    z0i = z0.astype(jnp.int32)
    lins, wts = [], []
    for dx in (0, 1):
        for dy in (0, 1):
            for dz in (0, 1):
                ix = jnp.clip(x0i + dx, 0, _NX - 1)
                iy = jnp.clip(y0i + dy, 0, _NY - 1)
                iz = jnp.clip(z0i + dz, 0, _NZ - 1)
                valid = valid & (x0i + dx >= 0) & (x0i + dx < _NX)
                valid = valid & (y0i + dy >= 0) & (y0i + dy < _NY)
                valid = valid & (z0i + dz >= 0) & (z0i + dz < _NZ)
                wx = fx if dx == 1 else 1.0 - fx
                wy = fy if dy == 1 else 1.0 - fy
                wz = fz if dz == 1 else 1.0 - fz
                w = wx * wy * wz * conf * valid.astype(jnp.float32)
                lins.append(iz * (_NY * _NX) + iy * _NX + ix)
                wts.append(w)
    lin_ref[0] = jnp.concatenate(lins, axis=0)
    wt_ref[0] = jnp.concatenate(wts, axis=0)
    base = vv * hw + bb * bhw
    pt = base + jax.lax.broadcasted_iota(jnp.int32, (8, bhw), 1)
    pt_ref[0] = pt


def _feat_t_kernel(feat_ref, out_ref):
    out_ref[0] = feat_ref[0].T


def _norm_mm_kernel(vol_ref, volw_ref, w_ref, b_ref, out_ref):
    f = vol_ref[...]
    wsum = volw_ref[...]
    nf = f / jnp.maximum(wsum, 1e-6)
    out = jax.lax.dot_general(
        w_ref[...], nf, (((1,), (1,)), ((), ())),
        preferred_element_type=jnp.float32)
    out_ref[...] = out + b_ref[...]


def _stage1(xyz, conf, feat):
    v, _, h4, w4 = xyz.shape[0], xyz.shape[1], xyz.shape[2], xyz.shape[3]
    hw = h4 * w4
    bhw = 1408
    nb = hw // bhw
    xyz3 = xyz.reshape(v, 3, hw)
    conf3 = conf.reshape(v, 1, hw)
    lin, wt, pt = pl.pallas_call(
        functools.partial(_idx_weight_kernel, bhw=bhw, hw=hw),
        grid=(v, nb),
        in_specs=[
            pl.BlockSpec((1, 3, bhw), lambda i, j: (i, 0, j)),
            pl.BlockSpec((1, 1, bhw), lambda i, j: (i, 0, j)),
        ],
        out_specs=[
            pl.BlockSpec((1, 8, bhw), lambda i, j: (i * nb + j, 0, 0)),
            pl.BlockSpec((1, 8, bhw), lambda i, j: (i * nb + j, 0, 0)),
            pl.BlockSpec((1, 8, bhw), lambda i, j: (i * nb + j, 0, 0)),
        ],
        out_shape=[
            jax.ShapeDtypeStruct((v * nb, 8, bhw), jnp.int32),
            jax.ShapeDtypeStruct((v * nb, 8, bhw), jnp.float32),
            jax.ShapeDtypeStruct((v * nb, 8, bhw), jnp.int32),
        ],
    )(xyz3, conf3)
    feat3 = feat.reshape(v, _C, hw)
    feat_t = pl.pallas_call(
        _feat_t_kernel,
        grid=(v, nb),
        in_specs=[pl.BlockSpec((1, _C, bhw), lambda i, j: (i, 0, j))],
        out_specs=pl.BlockSpec((1, bhw, _C), lambda i, j: (i * nb + j, 0, 0)),
        out_shape=jax.ShapeDtypeStruct((v * nb, bhw, _C), jnp.float32),
    )(feat3)
    n = v * hw
    lin8 = jnp.transpose(lin, (1, 0, 2)).reshape(8, n)
    wt8 = jnp.transpose(wt, (1, 0, 2)).reshape(8, n)
    return lin8, wt8, pt.reshape(-1), feat_t.reshape(n, _C)


def _stage3(vol, vol_w, W, bias):
    bv = 1536
    out = pl.pallas_call(
        _norm_mm_kernel,
        grid=(_NVP // bv,),
        in_specs=[
            pl.BlockSpec((bv, _C), lambda i: (i, 0)),
            pl.BlockSpec((bv, 1), lambda i: (i, 0)),
            pl.BlockSpec((_C, _C), lambda i: (0, 0)),
            pl.BlockSpec((_C, 1), lambda i: (0, 0)),
        ],
        out_specs=pl.BlockSpec((_C, bv), lambda i: (0, i)),
        out_shape=jax.ShapeDtypeStruct((_C, _NVP), jnp.float32),
    )(vol, vol_w, W, bias.reshape(_C, 1))
    return out[:, :_NV]


def _scatter_stage(lin8, wt8, feat_t):
    vol = jnp.zeros((_NVP, _C), jnp.float32)
    volw = jnp.zeros((_NVP, 1), jnp.float32)
    for k in range(8):
        w = wt8[k][:, None]
        vol = vol.at[lin8[k]].add(feat_t * w)
        volw = volw.at[lin8[k]].add(w)
    return vol, volw


def kernel(feat_1_4, xyz_1_4, conf_1_4, W, bias):
    b, t, v, c, h4, w4 = feat_1_4.shape
    feat = feat_1_4.reshape(v, c, h4, w4)
    xyz = xyz_1_4.reshape(v, 3, h4, w4)
    conf = conf_1_4.reshape(v, h4, w4)
    lin, wt, pt, feat_t = _stage1(xyz, conf, feat)
    del pt
    vol, vol_w = _scatter_stage(lin, wt, feat_t)
    out = _stage3(vol, vol_w, W, bias)
    return out.reshape(b, t, c, _NZ, _NY, _NX)
